# baseline (speedup 1.0000x reference)
import functools

import jax
import jax.numpy as jnp
from jax.experimental import pallas as pl
from jax.experimental.pallas import tpu as pltpu

_NUM_EXPERTS = 64
_BLOCK_M = 1024


def _top2_softmax(logits):
    v1 = jnp.max(logits, axis=-1, keepdims=True)
    iota = jax.lax.broadcasted_iota(jnp.int32, logits.shape, 1)
    idx1 = jnp.min(
        jnp.where(logits == v1, iota, _NUM_EXPERTS), axis=-1, keepdims=True
    )
    v2 = jnp.max(jnp.where(iota == idx1, -jnp.inf, logits), axis=-1, keepdims=True)
    e2 = jnp.exp(v2 - v1)
    denom = 1.0 + e2
    return jnp.concatenate([1.0 / denom, e2 / denom], axis=-1)


def _gating_body(xa_ref, xb_ref, w_ref, o_ref):
    w = w_ref[...]
    dims = (((1,), (1,)), ((), ()))
    la = jax.lax.dot_general(
        xa_ref[...], w, dims, preferred_element_type=jnp.float32
    )
    o_ref[0] = _top2_softmax(la)
    lb = jax.lax.dot_general(
        xb_ref[...], w, dims, preferred_element_type=jnp.float32
    )
    o_ref[1] = _top2_softmax(lb)


@functools.partial(jax.jit, static_argnames=("interpret",))
def kernel(x, W_g, interpret=False):
    n_tokens, dim = x.shape
    half_blocks = n_tokens // (2 * _BLOCK_M)
    out = pl.pallas_call(
        _gating_body,
        grid=(half_blocks,),
        in_specs=[
            pl.BlockSpec((_BLOCK_M, dim), lambda i: (i, 0)),
            pl.BlockSpec((_BLOCK_M, dim), lambda i, h=half_blocks: (i + h, 0)),
            pl.BlockSpec((_NUM_EXPERTS, dim), lambda i: (0, 0)),
        ],
        out_specs=pl.BlockSpec((2, _BLOCK_M, 2), lambda i: (0, i, 0)),
        out_shape=jax.ShapeDtypeStruct((2, n_tokens // 2, 2), jnp.float32),
        compiler_params=pltpu.CompilerParams(
            dimension_semantics=("arbitrary",),
            vmem_limit_bytes=64 * 1024 * 1024,
        ),
        interpret=interpret,
    )(x, x, W_g)
    return out.reshape(n_tokens, 2)


# speedup vs baseline: 1.0100x; 1.0100x over previous
import functools

import jax
import jax.numpy as jnp
from jax.experimental import pallas as pl
from jax.experimental.pallas import tpu as pltpu

_NUM_EXPERTS = 64
_BLOCK_M = 1024


def _top2_softmax(logits):
    v1 = jnp.max(logits, axis=-1, keepdims=True)
    iota = jax.lax.broadcasted_iota(jnp.int32, logits.shape, 1)
    idx1 = jnp.min(
        jnp.where(logits == v1, iota, _NUM_EXPERTS), axis=-1, keepdims=True
    )
    v2 = jnp.max(jnp.where(iota == idx1, -jnp.inf, logits), axis=-1, keepdims=True)
    e2 = jnp.exp(v2 - v1)
    denom = 1.0 + e2
    return jnp.concatenate([1.0 / denom, e2 / denom], axis=-1)


def _gating_body(xa_ref, xb_ref, w_ref, oa_ref, ob_ref):
    w = w_ref[...]
    dims = (((1,), (1,)), ((), ()))
    la = jax.lax.dot_general(
        xa_ref[...], w, dims, preferred_element_type=jnp.float32
    )
    oa_ref[...] = _top2_softmax(la)
    lb = jax.lax.dot_general(
        xb_ref[...], w, dims, preferred_element_type=jnp.float32
    )
    ob_ref[...] = _top2_softmax(lb)


@functools.partial(jax.jit, static_argnames=("interpret",))
def kernel(x, W_g, interpret=False):
    n_tokens, dim = x.shape
    half_blocks = n_tokens // (2 * _BLOCK_M)
    grid = (half_blocks,)
    out_a, out_b = pl.pallas_call(
        _gating_body,
        grid=grid,
        in_specs=[
            pl.BlockSpec((_BLOCK_M, dim), lambda i: (i, 0)),
            pl.BlockSpec((_BLOCK_M, dim), lambda i, h=half_blocks: (i + h, 0)),
            pl.BlockSpec((_NUM_EXPERTS, dim), lambda i: (0, 0)),
        ],
        out_specs=[
            pl.BlockSpec((_BLOCK_M, 2), lambda i: (i, 0)),
            pl.BlockSpec((_BLOCK_M, 2), lambda i: (i, 0)),
        ],
        out_shape=[
            jax.ShapeDtypeStruct((n_tokens // 2, 2), jnp.float32),
            jax.ShapeDtypeStruct((n_tokens // 2, 2), jnp.float32),
        ],
        compiler_params=pltpu.CompilerParams(
            dimension_semantics=("arbitrary",),
            vmem_limit_bytes=64 * 1024 * 1024,
        ),
        interpret=interpret,
    )(x, x, W_g)
    return jnp.concatenate([out_a, out_b], axis=0)
